# baseline (speedup 1.0000x reference)
import jax
import jax.numpy as jnp
from jax import lax
from jax.experimental import pallas as pl
from jax.experimental.pallas import tpu as pltpu
from jax.experimental.pallas import tpu_sc as plsc

N = 10000
E = 320000
D = 128
NC, NS, L = 2, 16, 16
NW = NC * NS
NP = 12288
RPW = NP // NW
RPT = NP // NS

CHA = 79
CHC = 82

_MESH = plsc.VectorSubcoreMesh(
    core_axis_name="c", subcore_axis_name="s", num_cores=NC, num_subcores=NS)


def _wid():
    return lax.axis_index("s") * NC + lax.axis_index("c")


def _hist_body(dst_hbm, val_hbm, zf_hbm, hist_hbm, dst_v, val_v, bounce_v,
               sem_a, hist_sh):
    c = lax.axis_index("c")
    s = lax.axis_index("s")
    w = _wid()

    @pl.when(s == 0)
    def _():
        pltpu.sync_copy(zf_hbm, hist_sh)

    pltpu.sync_copy(dst_hbm.at[w], dst_v)
    pltpu.sync_copy(val_hbm.at[w], val_v)
    plsc.subcore_barrier()

    def body(j, carry):
        pltpu.async_copy(val_v.at[j], hist_sh.at[dst_v.at[j]], sem_a,
                         add=True)
        return carry

    lax.fori_loop(0, CHA, body, 0)

    def drain(j, carry):
        pltpu.make_async_copy(val_v.at[j], hist_sh.at[dst_v.at[j]],
                              sem_a).wait()
        return carry

    lax.fori_loop(0, CHA, drain, 0)
    plsc.subcore_barrier()

    @pl.when(s == 0)
    def _():
        pltpu.sync_copy(hist_sh, bounce_v)
        pltpu.sync_copy(bounce_v, hist_hbm.at[c])


def _make_hist():
    return pl.kernel(
        _hist_body,
        out_type=jax.ShapeDtypeStruct((NC, NP), jnp.float32),
        mesh=_MESH,
        scratch_types=[
            pltpu.VMEM((CHA, 128), jnp.int32),
            pltpu.VMEM((CHA, 128), jnp.float32),
            pltpu.VMEM((NP,), jnp.float32),
            pltpu.SemaphoreType.DMA,
            pltpu.VMEM_SHARED((NP,), jnp.float32),
        ],
    )


def _feat_body(hist_hbm, x_hbm, feat_hbm, normb_hbm, h0_v, h1_v, norm_v, x_v,
               nb_v):
    w = _wid()
    base = w * RPW
    pltpu.sync_copy(hist_hbm.at[0].at[pl.ds(base, RPW)], h0_v)
    pltpu.sync_copy(hist_hbm.at[1].at[pl.ds(base, RPW)], h1_v)
    pltpu.sync_copy(x_hbm.at[pl.ds(base * D, RPW * D)], x_v)

    def nbody(v, carry):
        deg = h0_v[pl.ds(v * L, L)] + h1_v[pl.ds(v * L, L)] + 1.0
        i = lax.bitcast_convert_type(deg, jnp.int32)
        i = 0x5F3759DF - lax.shift_right_logical(i, 1)
        y = lax.bitcast_convert_type(i, jnp.float32)
        hh = deg * 0.5
        y = y * (1.5 - hh * y * y)
        y = y * (1.5 - hh * y * y)
        y = y * (1.5 - hh * y * y)
        norm_v[pl.ds(v * L, L)] = y
        return carry

    lax.fori_loop(0, RPW // L, nbody, 0)

    def gbody(g, carry):
        nv = norm_v[pl.ds(g * L, L)]
        for lane in range(L):
            idx = (jnp.zeros((L,), jnp.int32) + lane)[:, None]
            nb = lax.gather(
                nv, idx,
                lax.GatherDimensionNumbers(offset_dims=(),
                                           collapsed_slice_dims=(0,),
                                           start_index_map=(0,)),
                slice_sizes=(1,),
                mode=lax.GatherScatterMode.PROMISE_IN_BOUNDS)
            for k in range(D // L):
                off = (g * L + lane) * D + k * L
                x_v[pl.ds(off, L)] = x_v[pl.ds(off, L)] * nb
                nb_v[pl.ds(off, L)] = nb
        return carry

    lax.fori_loop(0, RPW // L, gbody, 0)

    pltpu.sync_copy(x_v, feat_hbm.at[pl.ds(base * D, RPW * D)])
    pltpu.sync_copy(nb_v, normb_hbm.at[pl.ds(base * D, RPW * D)])


def _make_feat():
    return pl.kernel(
        _feat_body,
        out_type=(jax.ShapeDtypeStruct((NP * D,), jnp.float32),
                  jax.ShapeDtypeStruct((NP * D,), jnp.float32)),
        mesh=_MESH,
        scratch_types=[
            pltpu.VMEM((RPW,), jnp.float32),
            pltpu.VMEM((RPW,), jnp.float32),
            pltpu.VMEM((RPW,), jnp.float32),
            pltpu.VMEM((RPW * D,), jnp.float32),
            pltpu.VMEM((RPW * D,), jnp.float32),
        ],
    )


DH = D // 2
NBUF = 4
PF = 2
CHT = 168


def _agg_body(src_hbm, dst_hbm, feat2_hbm, z_hbm, pout_hbm,
              src_v, dst_v, bufs, sem_g, sem_s, agg_sh):
    c = lax.axis_index("c")
    s = lax.axis_index("s")

    @pl.when(s == 0)
    def _():
        pltpu.sync_copy(z_hbm, agg_sh)

    pltpu.sync_copy(src_hbm.at[s], src_v)
    pltpu.sync_copy(dst_hbm.at[s], dst_v)
    plsc.subcore_barrier()

    ftab = feat2_hbm.at[c]
    for n in range(PF):
        pltpu.async_copy(ftab.at[src_v.at[n]], bufs[n], sem_g[n])

    def body(t, carry):
        for b in range(NBUF):
            n = NBUF * t + b
            pltpu.make_async_copy(ftab.at[src_v.at[n]], bufs[b],
                                  sem_g[b]).wait()
            pltpu.async_copy(bufs[b], agg_sh.at[dst_v.at[n]], sem_s[b],
                             add=True)
            sf = (b + PF) % NBUF

            @pl.when(n + PF < CHT)
            def _():
                @pl.when(n >= PF)
                def _():
                    pltpu.make_async_copy(bufs[sf],
                                          agg_sh.at[dst_v.at[n - PF]],
                                          sem_s[sf]).wait()

                pltpu.async_copy(ftab.at[src_v.at[n + PF]], bufs[sf],
                                 sem_g[sf])
        return carry

    lax.fori_loop(0, CHT // NBUF, body, 0)
    for b in range(NBUF):
        pltpu.make_async_copy(bufs[b], agg_sh.at[dst_v.at[CHT - NBUF + b]],
                              sem_s[b]).wait()
    plsc.subcore_barrier()

    for q in range(RPT // 128):
        row0 = s * RPT + q * 128
        pltpu.sync_copy(agg_sh.at[pl.ds(row0, 128)], bufs[0])
        pltpu.sync_copy(bufs[0], pout_hbm.at[c].at[pl.ds(row0, 128)])


def _make_agg():
    return pl.kernel(
        _agg_body,
        out_type=jax.ShapeDtypeStruct((NC, NP, DH), jnp.float32),
        mesh=_MESH,
        scratch_types=[
            pltpu.VMEM((CHT, 128), jnp.int32),
            pltpu.VMEM((CHT, 128), jnp.int32),
            [pltpu.VMEM((128, DH), jnp.float32) for _ in range(NBUF)],
            [pltpu.SemaphoreType.DMA for _ in range(NBUF)],
            [pltpu.SemaphoreType.DMA for _ in range(NBUF)],
            pltpu.VMEM_SHARED((NP, DH), jnp.float32),
        ],
        compiler_params=pltpu.CompilerParams(use_tc_tiling_on_sc=False),
    )


_BD = 512


def _mm_body(p_ref, nb_ref, w_ref, o_ref):
    a = p_ref[...] * nb_ref[...]
    o_ref[...] = lax.dot_general(a, w_ref[...], (((1,), (1,)), ((), ())),
                                 preferred_element_type=jnp.float32)


def _matmul(p, normb, w):
    return pl.pallas_call(
        _mm_body,
        grid=(NP // _BD,),
        in_specs=[
            pl.BlockSpec((_BD, D), lambda i: (i, 0)),
            pl.BlockSpec((_BD, D), lambda i: (i, 0)),
            pl.BlockSpec((D, D), lambda i: (0, 0)),
        ],
        out_specs=pl.BlockSpec((_BD, D), lambda i: (i, 0)),
        out_shape=jax.ShapeDtypeStruct((NP, D), jnp.float32),
    )(p, normb, w)


NR = NP // 128
NP2 = NP + 128


def _pos_body(h_ref, u_ref, s_ref, o_ref):
    p = ((h_ref[0] + h_ref[1]) > 0.0).astype(jnp.float32)
    incl = lax.dot_general(p, u_ref[...], (((1,), (0,)), ((), ())),
                           precision=lax.Precision.HIGHEST,
                           preferred_element_type=jnp.float32)
    rows = incl[:, 127:128]
    rowpref = lax.dot_general(s_ref[...], rows, (((1,), (0,)), ((), ())),
                              precision=lax.Precision.HIGHEST,
                              preferred_element_type=jnp.float32)
    excl = incl - p + rowpref
    col = lax.broadcasted_iota(jnp.int32, (NR, 128), 1)
    o_ref[...] = jnp.where(p > 0.0, excl.astype(jnp.int32), NP + col)


def _positions(h3, u128, s96):
    return pl.pallas_call(
        _pos_body,
        grid=(1,),
        in_specs=[
            pl.BlockSpec((2, NR, 128), lambda i: (0, 0, 0)),
            pl.BlockSpec((128, 128), lambda i: (0, 0)),
            pl.BlockSpec((NR, NR), lambda i: (0, 0)),
        ],
        out_specs=pl.BlockSpec((NR, 128), lambda i: (0, 0)),
        out_shape=jax.ShapeDtypeStruct((NR, 128), jnp.int32),
    )(h3, u128, s96)


RPS = 8


def _gather_body(midx_hbm, vals_hbm, src2d_hbm, zi_hbm, fin_hbm, pidx_v,
                 pval_v, idx_v, rows_v, sem, dn_sh):
    s = lax.axis_index("s")
    w = _wid()

    @pl.when(s == 0)
    def _():
        pltpu.sync_copy(zi_hbm, dn_sh)

    @pl.when(s < NR // RPS)
    def _():
        pltpu.sync_copy(midx_hbm.at[pl.ds(s * RPS, RPS)], pidx_v)
        pltpu.sync_copy(vals_hbm.at[pl.ds(s * RPS, RPS)], pval_v)

    plsc.subcore_barrier()

    @pl.when(s < NR // RPS)
    def _():
        for r in range(RPS):
            pltpu.sync_copy(pval_v.at[r], dn_sh.at[pidx_v.at[r]])

    plsc.subcore_barrier()

    base = w * RPW
    pltpu.sync_copy(dn_sh.at[pl.ds(base, RPW)], idx_v)
    for off in (0, 128, 256):
        pltpu.async_copy(src2d_hbm.at[idx_v.at[pl.ds(off, 128)]],
                         rows_v, sem).wait()
        pltpu.sync_copy(rows_v, fin_hbm.at[pl.ds(base + off, 128)])


def _make_gather():
    return pl.kernel(
        _gather_body,
        out_type=jax.ShapeDtypeStruct((NP, D), jnp.float32),
        mesh=_MESH,
        scratch_types=[
            pltpu.VMEM((RPS, 128), jnp.int32),
            pltpu.VMEM((RPS, 128), jnp.int32),
            pltpu.VMEM((RPW,), jnp.int32),
            pltpu.VMEM((128, D), jnp.float32),
            pltpu.SemaphoreType.DMA,
            pltpu.VMEM_SHARED((NP2,), jnp.int32),
        ],
    )


def kernel(x, edge_index, W):
    src = edge_index[0].astype(jnp.int32)
    dst = edge_index[1].astype(jnp.int32)
    x = x.astype(jnp.float32)
    xp = jnp.pad(x, ((0, NP - N), (0, 0)))

    n_pad_a = NW * CHA * 128 - E
    pad_a = N + (jnp.arange(n_pad_a, dtype=jnp.int32) % (NP - N))
    dst_a = jnp.concatenate([dst, pad_a]).reshape(NW, CHA, 128)
    val_a = (jnp.arange(NW * CHA * 128) < E).astype(jnp.float32)
    val_a = val_a.reshape(NW, CHA, 128)

    loop = jnp.arange(N, dtype=jnp.int32)
    n_pad_c = NS * CHT * 128 - (E + N)
    pad_c = N + (jnp.arange(n_pad_c, dtype=jnp.int32) % (NP - N))
    src_c = jnp.concatenate([src, loop, pad_c]).reshape(NS, CHT, 128)
    dst_c = jnp.concatenate([dst, loop, pad_c]).reshape(NS, CHT, 128)

    zf = jnp.zeros((NP,), jnp.float32)
    z2 = jnp.zeros((NP, DH), jnp.float32)
    zi = jnp.concatenate([
        jnp.zeros((N,), jnp.int32),
        jnp.arange(NP2 - N, dtype=jnp.int32) % N,
    ])
    u128 = jnp.triu(jnp.ones((128, 128), jnp.float32))
    s96 = jnp.tril(jnp.ones((NR, NR), jnp.float32), k=-1)
    vals = jnp.arange(NP, dtype=jnp.int32).reshape(NR, 128)

    hist = _make_hist()(dst_a, val_a, zf)
    feat_flat, normb_flat = _make_feat()(hist, xp.reshape(-1))
    feat = feat_flat.reshape(NP, D)
    feat2 = jnp.stack([feat[:, :DH], feat[:, DH:]])
    part = _make_agg()(src_c, dst_c, feat2, z2)
    q = jnp.concatenate([part[0], part[1]], axis=1)
    out_d = _matmul(q, normb_flat.reshape(NP, D), W.astype(jnp.float32))
    midx = _positions(hist.reshape(2, NR, 128), u128, s96)
    fin = _make_gather()(midx, vals, out_d, zi)
    return fin[:N]


# speedup vs baseline: 1.1101x; 1.1101x over previous
import jax
import jax.numpy as jnp
from jax import lax
from jax.experimental import pallas as pl
from jax.experimental.pallas import tpu as pltpu
from jax.experimental.pallas import tpu_sc as plsc

N = 10000
E = 320000
D = 128
NC, NS, L = 2, 16, 16
NW = NC * NS
NP = 12288
RPW = NP // NW
RPT = NP // NS

CHA = 79
CHC = 82

_MESH = plsc.VectorSubcoreMesh(
    core_axis_name="c", subcore_axis_name="s", num_cores=NC, num_subcores=NS)


def _wid():
    return lax.axis_index("s") * NC + lax.axis_index("c")


def _hist_body(dst_hbm, val_hbm, zf_hbm, hist_hbm, dst_v, val_v, bounce_v,
               sem_a, hist_sh):
    c = lax.axis_index("c")
    s = lax.axis_index("s")
    w = _wid()

    @pl.when(s == 0)
    def _():
        pltpu.sync_copy(zf_hbm, hist_sh)

    pltpu.sync_copy(dst_hbm.at[w], dst_v)
    pltpu.sync_copy(val_hbm.at[w], val_v)
    plsc.subcore_barrier()

    def body(j, carry):
        pltpu.async_copy(val_v.at[j], hist_sh.at[dst_v.at[j]], sem_a,
                         add=True)
        return carry

    lax.fori_loop(0, CHA, body, 0)

    def drain(j, carry):
        pltpu.make_async_copy(val_v.at[j], hist_sh.at[dst_v.at[j]],
                              sem_a).wait()
        return carry

    lax.fori_loop(0, CHA, drain, 0)
    plsc.subcore_barrier()

    @pl.when(s == 0)
    def _():
        pltpu.sync_copy(hist_sh, bounce_v)
        pltpu.sync_copy(bounce_v, hist_hbm.at[c])


def _make_hist():
    return pl.kernel(
        _hist_body,
        out_type=jax.ShapeDtypeStruct((NC, NP), jnp.float32),
        mesh=_MESH,
        scratch_types=[
            pltpu.VMEM((CHA, 128), jnp.int32),
            pltpu.VMEM((CHA, 128), jnp.float32),
            pltpu.VMEM((NP,), jnp.float32),
            pltpu.SemaphoreType.DMA,
            pltpu.VMEM_SHARED((NP,), jnp.float32),
        ],
    )


def _feat_body(hist_hbm, x_hbm, feat_hbm, normb_hbm, h0_v, h1_v, norm_v, x_v,
               nb_v):
    w = _wid()
    base = w * RPW
    pltpu.sync_copy(hist_hbm.at[0].at[pl.ds(base, RPW)], h0_v)
    pltpu.sync_copy(hist_hbm.at[1].at[pl.ds(base, RPW)], h1_v)
    pltpu.sync_copy(x_hbm.at[pl.ds(base * D, RPW * D)], x_v)

    def nbody(v, carry):
        deg = h0_v[pl.ds(v * L, L)] + h1_v[pl.ds(v * L, L)] + 1.0
        i = lax.bitcast_convert_type(deg, jnp.int32)
        i = 0x5F3759DF - lax.shift_right_logical(i, 1)
        y = lax.bitcast_convert_type(i, jnp.float32)
        hh = deg * 0.5
        y = y * (1.5 - hh * y * y)
        y = y * (1.5 - hh * y * y)
        y = y * (1.5 - hh * y * y)
        norm_v[pl.ds(v * L, L)] = y
        return carry

    lax.fori_loop(0, RPW // L, nbody, 0)

    def gbody(g, carry):
        nv = norm_v[pl.ds(g * L, L)]
        for lane in range(L):
            idx = (jnp.zeros((L,), jnp.int32) + lane)[:, None]
            nb = lax.gather(
                nv, idx,
                lax.GatherDimensionNumbers(offset_dims=(),
                                           collapsed_slice_dims=(0,),
                                           start_index_map=(0,)),
                slice_sizes=(1,),
                mode=lax.GatherScatterMode.PROMISE_IN_BOUNDS)
            for k in range(D // L):
                off = (g * L + lane) * D + k * L
                x_v[pl.ds(off, L)] = x_v[pl.ds(off, L)] * nb
                nb_v[pl.ds(off, L)] = nb
        return carry

    lax.fori_loop(0, RPW // L, gbody, 0)

    pltpu.sync_copy(x_v, feat_hbm.at[pl.ds(base * D, RPW * D)])
    pltpu.sync_copy(nb_v, normb_hbm.at[pl.ds(base * D, RPW * D)])


def _make_feat():
    return pl.kernel(
        _feat_body,
        out_type=(jax.ShapeDtypeStruct((NP * D,), jnp.float32),
                  jax.ShapeDtypeStruct((NP * D,), jnp.float32)),
        mesh=_MESH,
        scratch_types=[
            pltpu.VMEM((RPW,), jnp.float32),
            pltpu.VMEM((RPW,), jnp.float32),
            pltpu.VMEM((RPW,), jnp.float32),
            pltpu.VMEM((RPW * D,), jnp.float32),
            pltpu.VMEM((RPW * D,), jnp.float32),
        ],
    )


DH = D // 2
NBUF = 4
PF = 2
CHT = 168


def _agg_body(src_hbm, dst_hbm, feat2_hbm, z_hbm, pout_hbm,
              src_v, dst_v, bufs, sem_g, sem_s, agg_sh):
    c = lax.axis_index("c")
    s = lax.axis_index("s")

    @pl.when(s == 0)
    def _():
        pltpu.sync_copy(z_hbm, agg_sh)

    pltpu.sync_copy(src_hbm.at[s], src_v)
    pltpu.sync_copy(dst_hbm.at[s], dst_v)
    plsc.subcore_barrier()

    ftab = feat2_hbm.at[c]
    for n in range(NBUF):
        pltpu.async_copy(ftab.at[src_v.at[n]], bufs[n], sem_g[n])

    def body(t, carry):
        for b in range(NBUF):
            n = NBUF * t + b
            pltpu.make_async_copy(ftab.at[src_v.at[n]], bufs[b],
                                  sem_g[b]).wait()
            pltpu.sync_copy(bufs[b], agg_sh.at[dst_v.at[n]], add=True)

            @pl.when(n + NBUF < CHT)
            def _():
                pltpu.async_copy(ftab.at[src_v.at[n + NBUF]], bufs[b],
                                 sem_g[b])
        return carry

    lax.fori_loop(0, CHT // NBUF, body, 0)
    plsc.subcore_barrier()

    for q in range(RPT // 128):
        row0 = s * RPT + q * 128
        pltpu.sync_copy(agg_sh.at[pl.ds(row0, 128)], bufs[0])
        pltpu.sync_copy(bufs[0], pout_hbm.at[c].at[pl.ds(row0, 128)])


def _make_agg():
    return pl.kernel(
        _agg_body,
        out_type=jax.ShapeDtypeStruct((NC, NP, DH), jnp.float32),
        mesh=_MESH,
        scratch_types=[
            pltpu.VMEM((CHT, 128), jnp.int32),
            pltpu.VMEM((CHT, 128), jnp.int32),
            [pltpu.VMEM((128, DH), jnp.float32) for _ in range(NBUF)],
            [pltpu.SemaphoreType.DMA for _ in range(NBUF)],
            [pltpu.SemaphoreType.DMA for _ in range(NBUF)],
            pltpu.VMEM_SHARED((NP, DH), jnp.float32),
        ],
        compiler_params=pltpu.CompilerParams(use_tc_tiling_on_sc=False),
    )


_BD = 512


def _mm_body(p_ref, nb_ref, w_ref, o_ref):
    a = p_ref[...] * nb_ref[...]
    o_ref[...] = lax.dot_general(a, w_ref[...], (((1,), (1,)), ((), ())),
                                 preferred_element_type=jnp.float32)


def _matmul(p, normb, w):
    return pl.pallas_call(
        _mm_body,
        grid=(NP // _BD,),
        in_specs=[
            pl.BlockSpec((_BD, D), lambda i: (i, 0)),
            pl.BlockSpec((_BD, D), lambda i: (i, 0)),
            pl.BlockSpec((D, D), lambda i: (0, 0)),
        ],
        out_specs=pl.BlockSpec((_BD, D), lambda i: (i, 0)),
        out_shape=jax.ShapeDtypeStruct((NP, D), jnp.float32),
    )(p, normb, w)


NR = NP // 128
NP2 = NP + 128


def _pos_body(h_ref, u_ref, s_ref, o_ref):
    p = ((h_ref[0] + h_ref[1]) > 0.0).astype(jnp.float32)
    incl = lax.dot_general(p, u_ref[...], (((1,), (0,)), ((), ())),
                           precision=lax.Precision.HIGHEST,
                           preferred_element_type=jnp.float32)
    rows = incl[:, 127:128]
    rowpref = lax.dot_general(s_ref[...], rows, (((1,), (0,)), ((), ())),
                              precision=lax.Precision.HIGHEST,
                              preferred_element_type=jnp.float32)
    excl = incl - p + rowpref
    col = lax.broadcasted_iota(jnp.int32, (NR, 128), 1)
    o_ref[...] = jnp.where(p > 0.0, excl.astype(jnp.int32), NP + col)


def _positions(h3, u128, s96):
    return pl.pallas_call(
        _pos_body,
        grid=(1,),
        in_specs=[
            pl.BlockSpec((2, NR, 128), lambda i: (0, 0, 0)),
            pl.BlockSpec((128, 128), lambda i: (0, 0)),
            pl.BlockSpec((NR, NR), lambda i: (0, 0)),
        ],
        out_specs=pl.BlockSpec((NR, 128), lambda i: (0, 0)),
        out_shape=jax.ShapeDtypeStruct((NR, 128), jnp.int32),
    )(h3, u128, s96)


RPS = 8


def _gather_body(midx_hbm, vals_hbm, src2d_hbm, zi_hbm, fin_hbm, pidx_v,
                 pval_v, idx_v, rows_v, sem, dn_sh):
    s = lax.axis_index("s")
    w = _wid()

    @pl.when(s == 0)
    def _():
        pltpu.sync_copy(zi_hbm, dn_sh)

    @pl.when(s < NR // RPS)
    def _():
        pltpu.sync_copy(midx_hbm.at[pl.ds(s * RPS, RPS)], pidx_v)
        pltpu.sync_copy(vals_hbm.at[pl.ds(s * RPS, RPS)], pval_v)

    plsc.subcore_barrier()

    @pl.when(s < NR // RPS)
    def _():
        for r in range(RPS):
            pltpu.sync_copy(pval_v.at[r], dn_sh.at[pidx_v.at[r]])

    plsc.subcore_barrier()

    base = w * RPW
    pltpu.sync_copy(dn_sh.at[pl.ds(base, RPW)], idx_v)
    for off in (0, 128, 256):
        pltpu.async_copy(src2d_hbm.at[idx_v.at[pl.ds(off, 128)]],
                         rows_v, sem).wait()
        pltpu.sync_copy(rows_v, fin_hbm.at[pl.ds(base + off, 128)])


def _make_gather():
    return pl.kernel(
        _gather_body,
        out_type=jax.ShapeDtypeStruct((NP, D), jnp.float32),
        mesh=_MESH,
        scratch_types=[
            pltpu.VMEM((RPS, 128), jnp.int32),
            pltpu.VMEM((RPS, 128), jnp.int32),
            pltpu.VMEM((RPW,), jnp.int32),
            pltpu.VMEM((128, D), jnp.float32),
            pltpu.SemaphoreType.DMA,
            pltpu.VMEM_SHARED((NP2,), jnp.int32),
        ],
    )


def kernel(x, edge_index, W):
    src = edge_index[0].astype(jnp.int32)
    dst = edge_index[1].astype(jnp.int32)
    x = x.astype(jnp.float32)
    xp = jnp.pad(x, ((0, NP - N), (0, 0)))

    n_pad_a = NW * CHA * 128 - E
    pad_a = N + (jnp.arange(n_pad_a, dtype=jnp.int32) % (NP - N))
    dst_a = jnp.concatenate([dst, pad_a]).reshape(NW, CHA, 128)
    val_a = (jnp.arange(NW * CHA * 128) < E).astype(jnp.float32)
    val_a = val_a.reshape(NW, CHA, 128)

    loop = jnp.arange(N, dtype=jnp.int32)
    n_pad_c = NS * CHT * 128 - (E + N)
    pad_c = N + (jnp.arange(n_pad_c, dtype=jnp.int32) % (NP - N))
    src_c = jnp.concatenate([src, loop, pad_c]).reshape(NS, CHT, 128)
    dst_c = jnp.concatenate([dst, loop, pad_c]).reshape(NS, CHT, 128)

    zf = jnp.zeros((NP,), jnp.float32)
    z2 = jnp.zeros((NP, DH), jnp.float32)
    zi = jnp.concatenate([
        jnp.zeros((N,), jnp.int32),
        jnp.arange(NP2 - N, dtype=jnp.int32) % N,
    ])
    u128 = jnp.triu(jnp.ones((128, 128), jnp.float32))
    s96 = jnp.tril(jnp.ones((NR, NR), jnp.float32), k=-1)
    vals = jnp.arange(NP, dtype=jnp.int32).reshape(NR, 128)

    hist = _make_hist()(dst_a, val_a, zf)
    feat_flat, normb_flat = _make_feat()(hist, xp.reshape(-1))
    feat = feat_flat.reshape(NP, D)
    feat2 = jnp.stack([feat[:, :DH], feat[:, DH:]])
    part = _make_agg()(src_c, dst_c, feat2, z2)
    q = jnp.concatenate([part[0], part[1]], axis=1)
    out_d = _matmul(q, normb_flat.reshape(NP, D), W.astype(jnp.float32))
    midx = _positions(hist.reshape(2, NR, 128), u128, s96)
    fin = _make_gather()(midx, vals, out_d, zi)
    return fin[:N]


# speedup vs baseline: 1.1300x; 1.0180x over previous
import jax
import jax.numpy as jnp
from jax import lax
from jax.experimental import pallas as pl
from jax.experimental.pallas import tpu as pltpu
from jax.experimental.pallas import tpu_sc as plsc

N = 10000
E = 320000
D = 128
NC, NS, L = 2, 16, 16
NW = NC * NS
NP = 12288
RPW = NP // NW
RPT = NP // NS

CHA = 79
CHC = 82

_MESH = plsc.VectorSubcoreMesh(
    core_axis_name="c", subcore_axis_name="s", num_cores=NC, num_subcores=NS)


def _wid():
    return lax.axis_index("s") * NC + lax.axis_index("c")


CHA2 = 157
GR = 128


def _bcast(nv, lane):
    idx = (jnp.zeros((L,), jnp.int32) + lane)[:, None]
    return lax.gather(
        nv, idx,
        lax.GatherDimensionNumbers(offset_dims=(),
                                   collapsed_slice_dims=(0,),
                                   start_index_map=(0,)),
        slice_sizes=(1,),
        mode=lax.GatherScatterMode.PROMISE_IN_BOUNDS)


def _prep_body(dst_hbm, val_hbm, zf_hbm, x_hbm,
               hist_hbm, feat2_hbm, normb_hbm,
               dst_v, val_v, h_v, norm_v, x_v, xa_v, xb_v, nb_v, bounce_v,
               sem_a, hist_sh):
    c = lax.axis_index("c")
    s = lax.axis_index("s")
    w = _wid()

    @pl.when(s == 0)
    def _():
        pltpu.sync_copy(zf_hbm, hist_sh)

    pltpu.sync_copy(dst_hbm.at[s], dst_v)
    pltpu.sync_copy(val_hbm, val_v)
    plsc.subcore_barrier()

    def body(j, carry):
        pltpu.async_copy(val_v.at[j], hist_sh.at[dst_v.at[j]], sem_a,
                         add=True)
        return carry

    lax.fori_loop(0, CHA2, body, 0)

    def drain(j, carry):
        pltpu.make_async_copy(val_v.at[j], hist_sh.at[dst_v.at[j]],
                              sem_a).wait()
        return carry

    lax.fori_loop(0, CHA2, drain, 0)
    plsc.subcore_barrier()

    base = w * RPW
    pltpu.sync_copy(hist_sh.at[pl.ds(base, RPW)], h_v)

    def nbody(v, carry):
        deg = h_v[pl.ds(v * L, L)] + 1.0
        i = lax.bitcast_convert_type(deg, jnp.int32)
        i = 0x5F3759DF - lax.shift_right_logical(i, 1)
        y = lax.bitcast_convert_type(i, jnp.float32)
        hh = deg * 0.5
        y = y * (1.5 - hh * y * y)
        y = y * (1.5 - hh * y * y)
        y = y * (1.5 - hh * y * y)
        norm_v[pl.ds(v * L, L)] = y
        return carry

    lax.fori_loop(0, RPW // L, nbody, 0)

    for g in range(RPW // GR):
        row0 = base + g * GR
        pltpu.sync_copy(x_hbm.at[pl.ds(row0 * D, GR * D)], x_v)

        def gbody(g16, carry):
            nv = norm_v[pl.ds(g * GR + g16 * L, L)]
            for lane in range(L):
                nb = _bcast(nv, lane)
                r = g16 * L + lane
                for k in range(D // L):
                    xx = x_v[pl.ds(r * D + k * L, L)] * nb
                    if k < DH // L:
                        xa_v[r, pl.ds(k * L, L)] = xx
                    else:
                        xb_v[r, pl.ds((k - DH // L) * L, L)] = xx
                    nb_v[pl.ds(r * D + k * L, L)] = nb
            return carry

        lax.fori_loop(0, GR // L, gbody, 0)
        pltpu.sync_copy(xa_v, feat2_hbm.at[0].at[pl.ds(row0, GR)])
        pltpu.sync_copy(xb_v, feat2_hbm.at[1].at[pl.ds(row0, GR)])
        pltpu.sync_copy(nb_v, normb_hbm.at[pl.ds(row0 * D, GR * D)])

    @pl.when(jnp.logical_and(s == 0, c == 0))
    def _():
        pltpu.sync_copy(hist_sh, bounce_v)
        pltpu.sync_copy(bounce_v, hist_hbm)


def _make_prep():
    return pl.kernel(
        _prep_body,
        out_type=(jax.ShapeDtypeStruct((NP,), jnp.float32),
                  jax.ShapeDtypeStruct((2, NP, DH), jnp.float32),
                  jax.ShapeDtypeStruct((NP * D,), jnp.float32)),
        mesh=_MESH,
        scratch_types=[
            pltpu.VMEM((CHA2, 128), jnp.int32),
            pltpu.VMEM((CHA2, 128), jnp.float32),
            pltpu.VMEM((RPW,), jnp.float32),
            pltpu.VMEM((RPW,), jnp.float32),
            pltpu.VMEM((GR * D,), jnp.float32),
            pltpu.VMEM((GR, DH), jnp.float32),
            pltpu.VMEM((GR, DH), jnp.float32),
            pltpu.VMEM((GR * D,), jnp.float32),
            pltpu.VMEM((NP,), jnp.float32),
            pltpu.SemaphoreType.DMA,
            pltpu.VMEM_SHARED((NP,), jnp.float32),
        ],
        compiler_params=pltpu.CompilerParams(use_tc_tiling_on_sc=False),
    )


DH = D // 2
NBUF = 4
PF = 2
CHT = 168


def _agg_body(src_hbm, dst_hbm, feat2_hbm, z_hbm, pout_hbm,
              src_v, dst_v, bufs, sem_g, sem_s, agg_sh):
    c = lax.axis_index("c")
    s = lax.axis_index("s")

    @pl.when(s == 0)
    def _():
        pltpu.sync_copy(z_hbm, agg_sh)

    pltpu.sync_copy(src_hbm.at[s], src_v)
    pltpu.sync_copy(dst_hbm.at[s], dst_v)
    plsc.subcore_barrier()

    ftab = feat2_hbm.at[c]
    for n in range(NBUF):
        pltpu.async_copy(ftab.at[src_v.at[n]], bufs[n], sem_g[n])

    def body(t, carry):
        for b in range(NBUF):
            n = NBUF * t + b
            pltpu.make_async_copy(ftab.at[src_v.at[n]], bufs[b],
                                  sem_g[b]).wait()
            pltpu.sync_copy(bufs[b], agg_sh.at[dst_v.at[n]], add=True)

            @pl.when(n + NBUF < CHT)
            def _():
                pltpu.async_copy(ftab.at[src_v.at[n + NBUF]], bufs[b],
                                 sem_g[b])
        return carry

    lax.fori_loop(0, CHT // NBUF, body, 0)
    plsc.subcore_barrier()

    for q in range(RPT // 128):
        row0 = s * RPT + q * 128
        pltpu.sync_copy(agg_sh.at[pl.ds(row0, 128)], bufs[0])
        pltpu.sync_copy(bufs[0], pout_hbm.at[c].at[pl.ds(row0, 128)])


def _make_agg():
    return pl.kernel(
        _agg_body,
        out_type=jax.ShapeDtypeStruct((NC, NP, DH), jnp.float32),
        mesh=_MESH,
        scratch_types=[
            pltpu.VMEM((CHT, 128), jnp.int32),
            pltpu.VMEM((CHT, 128), jnp.int32),
            [pltpu.VMEM((128, DH), jnp.float32) for _ in range(NBUF)],
            [pltpu.SemaphoreType.DMA for _ in range(NBUF)],
            [pltpu.SemaphoreType.DMA for _ in range(NBUF)],
            pltpu.VMEM_SHARED((NP, DH), jnp.float32),
        ],
        compiler_params=pltpu.CompilerParams(use_tc_tiling_on_sc=False),
    )


_BD = 512


def _mm_body(p_ref, nb_ref, w_ref, o_ref):
    a = p_ref[...] * nb_ref[...]
    o_ref[...] = lax.dot_general(a, w_ref[...], (((1,), (1,)), ((), ())),
                                 preferred_element_type=jnp.float32)


def _matmul(p, normb, w):
    return pl.pallas_call(
        _mm_body,
        grid=(NP // _BD,),
        in_specs=[
            pl.BlockSpec((_BD, D), lambda i: (i, 0)),
            pl.BlockSpec((_BD, D), lambda i: (i, 0)),
            pl.BlockSpec((D, D), lambda i: (0, 0)),
        ],
        out_specs=pl.BlockSpec((_BD, D), lambda i: (i, 0)),
        out_shape=jax.ShapeDtypeStruct((NP, D), jnp.float32),
    )(p, normb, w)


NR = NP // 128
NP2 = NP + 128


def _pos_body(h_ref, u_ref, s_ref, o_ref):
    p = (h_ref[...] > 0.0).astype(jnp.float32)
    incl = lax.dot_general(p, u_ref[...], (((1,), (0,)), ((), ())),
                           precision=lax.Precision.HIGHEST,
                           preferred_element_type=jnp.float32)
    rows = incl[:, 127:128]
    rowpref = lax.dot_general(s_ref[...], rows, (((1,), (0,)), ((), ())),
                              precision=lax.Precision.HIGHEST,
                              preferred_element_type=jnp.float32)
    excl = incl - p + rowpref
    col = lax.broadcasted_iota(jnp.int32, (NR, 128), 1)
    o_ref[...] = jnp.where(p > 0.0, excl.astype(jnp.int32), NP + col)


def _positions(h3, u128, s96):
    return pl.pallas_call(
        _pos_body,
        grid=(1,),
        in_specs=[
            pl.BlockSpec((NR, 128), lambda i: (0, 0)),
            pl.BlockSpec((128, 128), lambda i: (0, 0)),
            pl.BlockSpec((NR, NR), lambda i: (0, 0)),
        ],
        out_specs=pl.BlockSpec((NR, 128), lambda i: (0, 0)),
        out_shape=jax.ShapeDtypeStruct((NR, 128), jnp.int32),
    )(h3, u128, s96)


RPS = 8


def _gather_body(midx_hbm, vals_hbm, src2d_hbm, zi_hbm, fin_hbm, pidx_v,
                 pval_v, idx_v, rows_v, sem, dn_sh):
    s = lax.axis_index("s")
    w = _wid()

    @pl.when(s == 0)
    def _():
        pltpu.sync_copy(zi_hbm, dn_sh)

    @pl.when(s < NR // RPS)
    def _():
        pltpu.sync_copy(midx_hbm.at[pl.ds(s * RPS, RPS)], pidx_v)
        pltpu.sync_copy(vals_hbm.at[pl.ds(s * RPS, RPS)], pval_v)

    plsc.subcore_barrier()

    @pl.when(s < NR // RPS)
    def _():
        for r in range(RPS):
            pltpu.sync_copy(pval_v.at[r], dn_sh.at[pidx_v.at[r]])

    plsc.subcore_barrier()

    base = w * RPW
    pltpu.sync_copy(dn_sh.at[pl.ds(base, RPW)], idx_v)
    for off in (0, 128, 256):
        pltpu.async_copy(src2d_hbm.at[idx_v.at[pl.ds(off, 128)]],
                         rows_v, sem).wait()
        pltpu.sync_copy(rows_v, fin_hbm.at[pl.ds(base + off, 128)])


def _make_gather():
    return pl.kernel(
        _gather_body,
        out_type=jax.ShapeDtypeStruct((NP, D), jnp.float32),
        mesh=_MESH,
        scratch_types=[
            pltpu.VMEM((RPS, 128), jnp.int32),
            pltpu.VMEM((RPS, 128), jnp.int32),
            pltpu.VMEM((RPW,), jnp.int32),
            pltpu.VMEM((128, D), jnp.float32),
            pltpu.SemaphoreType.DMA,
            pltpu.VMEM_SHARED((NP2,), jnp.int32),
        ],
    )


def kernel(x, edge_index, W):
    src = edge_index[0].astype(jnp.int32)
    dst = edge_index[1].astype(jnp.int32)
    x = x.astype(jnp.float32)
    xp = jnp.pad(x, ((0, NP - N), (0, 0)))

    pad_a = (N + (jnp.arange(NS * (CHA2 * 128 - E // NS), dtype=jnp.int32)
                  % (NP - N))).reshape(NS, -1)
    dst_a = jnp.concatenate([dst.reshape(NS, E // NS), pad_a],
                            axis=1).reshape(NS, CHA2, 128)
    val_a = (jnp.arange(CHA2 * 128) < E // NS).astype(jnp.float32)
    val_a = val_a.reshape(CHA2, 128)

    loop = jnp.arange(N, dtype=jnp.int32)
    n_pad_c = NS * CHT * 128 - (E + N)
    pad_c = N + (jnp.arange(n_pad_c, dtype=jnp.int32) % (NP - N))
    src_c = jnp.concatenate([src, loop, pad_c]).reshape(NS, CHT, 128)
    dst_c = jnp.concatenate([dst, loop, pad_c]).reshape(NS, CHT, 128)

    zf = jnp.zeros((NP,), jnp.float32)
    z2 = jnp.zeros((NP, DH), jnp.float32)
    zi = jnp.concatenate([
        jnp.zeros((N,), jnp.int32),
        jnp.arange(NP2 - N, dtype=jnp.int32) % N,
    ])
    u128 = jnp.triu(jnp.ones((128, 128), jnp.float32))
    s96 = jnp.tril(jnp.ones((NR, NR), jnp.float32), k=-1)
    vals = jnp.arange(NP, dtype=jnp.int32).reshape(NR, 128)

    hist, feat2, normb_flat = _make_prep()(dst_a, val_a, zf, xp.reshape(-1))
    part = _make_agg()(src_c, dst_c, feat2, z2)
    q = jnp.concatenate([part[0], part[1]], axis=1)
    out_d = _matmul(q, normb_flat.reshape(NP, D), W.astype(jnp.float32))
    midx = _positions(hist.reshape(NR, 128), u128, s96)
    fin = _make_gather()(midx, vals, out_d, zi)
    return fin[:N]


# speedup vs baseline: 1.1312x; 1.0010x over previous
import jax
import jax.numpy as jnp
from jax import lax
from jax.experimental import pallas as pl
from jax.experimental.pallas import tpu as pltpu
from jax.experimental.pallas import tpu_sc as plsc

N = 10000
E = 320000
D = 128
NC, NS, L = 2, 16, 16
NW = NC * NS
NP = 12288
RPW = NP // NW
RPT = NP // NS

CHA = 79
CHC = 82

_MESH = plsc.VectorSubcoreMesh(
    core_axis_name="c", subcore_axis_name="s", num_cores=NC, num_subcores=NS)


def _wid():
    return lax.axis_index("s") * NC + lax.axis_index("c")


CHA2 = 157
GR = 128


def _bcast(nv, lane):
    idx = (jnp.zeros((L,), jnp.int32) + lane)[:, None]
    return lax.gather(
        nv, idx,
        lax.GatherDimensionNumbers(offset_dims=(),
                                   collapsed_slice_dims=(0,),
                                   start_index_map=(0,)),
        slice_sizes=(1,),
        mode=lax.GatherScatterMode.PROMISE_IN_BOUNDS)


def _prep_body(dst_hbm, val_hbm, zf_hbm, x_hbm,
               hist_hbm, feat2_hbm, normb_hbm,
               dst_v, val_v, h_v, norm_v, x_v, xa_v, xb_v, nb_v, bounce_v,
               sem_a, hist_sh):
    c = lax.axis_index("c")
    s = lax.axis_index("s")
    w = _wid()

    @pl.when(s == 0)
    def _():
        pltpu.sync_copy(zf_hbm, hist_sh)

    pltpu.sync_copy(dst_hbm.at[s], dst_v)
    pltpu.sync_copy(val_hbm, val_v)
    plsc.subcore_barrier()

    pltpu.sync_copy(val_v, hist_sh.at[dst_v], add=True)
    plsc.subcore_barrier()

    base = w * RPW
    pltpu.sync_copy(hist_sh.at[pl.ds(base, RPW)], h_v)

    def nbody(v, carry):
        deg = h_v[pl.ds(v * L, L)] + 1.0
        i = lax.bitcast_convert_type(deg, jnp.int32)
        i = 0x5F3759DF - lax.shift_right_logical(i, 1)
        y = lax.bitcast_convert_type(i, jnp.float32)
        hh = deg * 0.5
        y = y * (1.5 - hh * y * y)
        y = y * (1.5 - hh * y * y)
        y = y * (1.5 - hh * y * y)
        norm_v[pl.ds(v * L, L)] = y
        return carry

    lax.fori_loop(0, RPW // L, nbody, 0)

    for g in range(RPW // GR):
        row0 = base + g * GR
        pltpu.sync_copy(x_hbm.at[pl.ds(row0 * D, GR * D)], x_v)

        def gbody(g16, carry):
            nv = norm_v[pl.ds(g * GR + g16 * L, L)]
            for lane in range(L):
                nb = _bcast(nv, lane)
                r = g16 * L + lane
                for k in range(D // L):
                    xx = x_v[pl.ds(r * D + k * L, L)] * nb
                    if k < DH // L:
                        xa_v[r, pl.ds(k * L, L)] = xx
                    else:
                        xb_v[r, pl.ds((k - DH // L) * L, L)] = xx
                    nb_v[pl.ds(r * D + k * L, L)] = nb
            return carry

        lax.fori_loop(0, GR // L, gbody, 0)
        pltpu.sync_copy(xa_v, feat2_hbm.at[0].at[pl.ds(row0, GR)])
        pltpu.sync_copy(xb_v, feat2_hbm.at[1].at[pl.ds(row0, GR)])
        pltpu.sync_copy(nb_v, normb_hbm.at[pl.ds(row0 * D, GR * D)])

    @pl.when(jnp.logical_and(s == 0, c == 0))
    def _():
        pltpu.sync_copy(hist_sh, bounce_v)
        pltpu.sync_copy(bounce_v, hist_hbm)


def _make_prep():
    return pl.kernel(
        _prep_body,
        out_type=(jax.ShapeDtypeStruct((NP,), jnp.float32),
                  jax.ShapeDtypeStruct((2, NP, DH), jnp.float32),
                  jax.ShapeDtypeStruct((NP * D,), jnp.float32)),
        mesh=_MESH,
        scratch_types=[
            pltpu.VMEM((CHA2 * 128,), jnp.int32),
            pltpu.VMEM((CHA2 * 128,), jnp.float32),
            pltpu.VMEM((RPW,), jnp.float32),
            pltpu.VMEM((RPW,), jnp.float32),
            pltpu.VMEM((GR * D,), jnp.float32),
            pltpu.VMEM((GR, DH), jnp.float32),
            pltpu.VMEM((GR, DH), jnp.float32),
            pltpu.VMEM((GR * D,), jnp.float32),
            pltpu.VMEM((NP,), jnp.float32),
            pltpu.SemaphoreType.DMA,
            pltpu.VMEM_SHARED((NP,), jnp.float32),
        ],
        compiler_params=pltpu.CompilerParams(use_tc_tiling_on_sc=False),
    )


DH = D // 2
NBUF = 4
PF = 2
CHT = 168


def _agg_body(src_hbm, dst_hbm, feat2_hbm, z_hbm, pout_hbm,
              src_v, dst_v, bufs, sem_g, sem_s, agg_sh):
    c = lax.axis_index("c")
    s = lax.axis_index("s")

    @pl.when(s == 0)
    def _():
        pltpu.sync_copy(z_hbm, agg_sh)

    pltpu.sync_copy(src_hbm.at[s], src_v)
    pltpu.sync_copy(dst_hbm.at[s], dst_v)
    plsc.subcore_barrier()

    ftab = feat2_hbm.at[c]
    for n in range(NBUF):
        pltpu.async_copy(ftab.at[src_v.at[n]], bufs[n], sem_g[n])

    def body(t, carry):
        for b in range(NBUF):
            n = NBUF * t + b
            pltpu.make_async_copy(ftab.at[src_v.at[n]], bufs[b],
                                  sem_g[b]).wait()
            pltpu.sync_copy(bufs[b], agg_sh.at[dst_v.at[n]], add=True)

            @pl.when(n + NBUF < CHT)
            def _():
                pltpu.async_copy(ftab.at[src_v.at[n + NBUF]], bufs[b],
                                 sem_g[b])
        return carry

    lax.fori_loop(0, CHT // NBUF, body, 0)
    plsc.subcore_barrier()

    for q in range(RPT // 128):
        row0 = s * RPT + q * 128
        pltpu.sync_copy(agg_sh.at[pl.ds(row0, 128)], bufs[0])
        pltpu.sync_copy(bufs[0], pout_hbm.at[c].at[pl.ds(row0, 128)])


def _make_agg():
    return pl.kernel(
        _agg_body,
        out_type=jax.ShapeDtypeStruct((NC, NP, DH), jnp.float32),
        mesh=_MESH,
        scratch_types=[
            pltpu.VMEM((CHT, 128), jnp.int32),
            pltpu.VMEM((CHT, 128), jnp.int32),
            [pltpu.VMEM((128, DH), jnp.float32) for _ in range(NBUF)],
            [pltpu.SemaphoreType.DMA for _ in range(NBUF)],
            [pltpu.SemaphoreType.DMA for _ in range(NBUF)],
            pltpu.VMEM_SHARED((NP, DH), jnp.float32),
        ],
        compiler_params=pltpu.CompilerParams(use_tc_tiling_on_sc=False),
    )


_BD = 512


def _mm_body(p_ref, nb_ref, w_ref, o_ref):
    a = p_ref[...] * nb_ref[...]
    o_ref[...] = lax.dot_general(a, w_ref[...], (((1,), (1,)), ((), ())),
                                 preferred_element_type=jnp.float32)


def _matmul(p, normb, w):
    return pl.pallas_call(
        _mm_body,
        grid=(NP // _BD,),
        in_specs=[
            pl.BlockSpec((_BD, D), lambda i: (i, 0)),
            pl.BlockSpec((_BD, D), lambda i: (i, 0)),
            pl.BlockSpec((D, D), lambda i: (0, 0)),
        ],
        out_specs=pl.BlockSpec((_BD, D), lambda i: (i, 0)),
        out_shape=jax.ShapeDtypeStruct((NP, D), jnp.float32),
    )(p, normb, w)


NR = NP // 128
NP2 = NP + 128


def _pos_body(h_ref, u_ref, s_ref, o_ref):
    p = (h_ref[...] > 0.0).astype(jnp.float32)
    incl = lax.dot_general(p, u_ref[...], (((1,), (0,)), ((), ())),
                           precision=lax.Precision.HIGHEST,
                           preferred_element_type=jnp.float32)
    rows = incl[:, 127:128]
    rowpref = lax.dot_general(s_ref[...], rows, (((1,), (0,)), ((), ())),
                              precision=lax.Precision.HIGHEST,
                              preferred_element_type=jnp.float32)
    excl = incl - p + rowpref
    col = lax.broadcasted_iota(jnp.int32, (NR, 128), 1)
    o_ref[...] = jnp.where(p > 0.0, excl.astype(jnp.int32), NP + col)


def _positions(h3, u128, s96):
    return pl.pallas_call(
        _pos_body,
        grid=(1,),
        in_specs=[
            pl.BlockSpec((NR, 128), lambda i: (0, 0)),
            pl.BlockSpec((128, 128), lambda i: (0, 0)),
            pl.BlockSpec((NR, NR), lambda i: (0, 0)),
        ],
        out_specs=pl.BlockSpec((NR, 128), lambda i: (0, 0)),
        out_shape=jax.ShapeDtypeStruct((NR, 128), jnp.int32),
    )(h3, u128, s96)


RPS = 8


def _gather_body(midx_hbm, vals_hbm, src2d_hbm, zi_hbm, fin_hbm, pidx_v,
                 pval_v, idx_v, rows_v, sem, dn_sh):
    s = lax.axis_index("s")
    w = _wid()

    @pl.when(s == 0)
    def _():
        pltpu.sync_copy(zi_hbm, dn_sh)

    @pl.when(s < NR // RPS)
    def _():
        pltpu.sync_copy(midx_hbm.at[pl.ds(s * RPS, RPS)], pidx_v)
        pltpu.sync_copy(vals_hbm.at[pl.ds(s * RPS, RPS)], pval_v)

    plsc.subcore_barrier()

    @pl.when(s < NR // RPS)
    def _():
        for r in range(RPS):
            pltpu.sync_copy(pval_v.at[r], dn_sh.at[pidx_v.at[r]])

    plsc.subcore_barrier()

    base = w * RPW
    pltpu.sync_copy(dn_sh.at[pl.ds(base, RPW)], idx_v)
    for off in (0, 128, 256):
        pltpu.async_copy(src2d_hbm.at[idx_v.at[pl.ds(off, 128)]],
                         rows_v, sem).wait()
        pltpu.sync_copy(rows_v, fin_hbm.at[pl.ds(base + off, 128)])


def _make_gather():
    return pl.kernel(
        _gather_body,
        out_type=jax.ShapeDtypeStruct((NP, D), jnp.float32),
        mesh=_MESH,
        scratch_types=[
            pltpu.VMEM((RPS, 128), jnp.int32),
            pltpu.VMEM((RPS, 128), jnp.int32),
            pltpu.VMEM((RPW,), jnp.int32),
            pltpu.VMEM((128, D), jnp.float32),
            pltpu.SemaphoreType.DMA,
            pltpu.VMEM_SHARED((NP2,), jnp.int32),
        ],
    )


def kernel(x, edge_index, W):
    src = edge_index[0].astype(jnp.int32)
    dst = edge_index[1].astype(jnp.int32)
    x = x.astype(jnp.float32)
    xp = jnp.pad(x, ((0, NP - N), (0, 0)))

    pad_a = (N + (jnp.arange(NS * (CHA2 * 128 - E // NS), dtype=jnp.int32)
                  % (NP - N))).reshape(NS, -1)
    dst_a = jnp.concatenate([dst.reshape(NS, E // NS), pad_a], axis=1)
    val_a = (jnp.arange(CHA2 * 128) < E // NS).astype(jnp.float32)

    loop = jnp.arange(N, dtype=jnp.int32)
    n_pad_c = NS * CHT * 128 - (E + N)
    pad_c = N + (jnp.arange(n_pad_c, dtype=jnp.int32) % (NP - N))
    src_c = jnp.concatenate([src, loop, pad_c]).reshape(NS, CHT, 128)
    dst_c = jnp.concatenate([dst, loop, pad_c]).reshape(NS, CHT, 128)

    zf = jnp.zeros((NP,), jnp.float32)
    z2 = jnp.zeros((NP, DH), jnp.float32)
    zi = jnp.concatenate([
        jnp.zeros((N,), jnp.int32),
        jnp.arange(NP2 - N, dtype=jnp.int32) % N,
    ])
    u128 = jnp.triu(jnp.ones((128, 128), jnp.float32))
    s96 = jnp.tril(jnp.ones((NR, NR), jnp.float32), k=-1)
    vals = jnp.arange(NP, dtype=jnp.int32).reshape(NR, 128)

    hist, feat2, normb_flat = _make_prep()(dst_a, val_a, zf, xp.reshape(-1))
    part = _make_agg()(src_c, dst_c, feat2, z2)
    q = jnp.concatenate([part[0], part[1]], axis=1)
    out_d = _matmul(q, normb_flat.reshape(NP, D), W.astype(jnp.float32))
    midx = _positions(hist.reshape(NR, 128), u128, s96)
    fin = _make_gather()(midx, vals, out_d, zi)
    return fin[:N]


# speedup vs baseline: 1.1947x; 1.0562x over previous
import jax
import jax.numpy as jnp
from jax import lax
from jax.experimental import pallas as pl
from jax.experimental.pallas import tpu as pltpu
from jax.experimental.pallas import tpu_sc as plsc

N = 10000
E = 320000
D = 128
NC, NS, L = 2, 16, 16
NW = NC * NS
NP = 12288
RPW = NP // NW
RPT = NP // NS

CHA = 79
CHC = 82

_MESH = plsc.VectorSubcoreMesh(
    core_axis_name="c", subcore_axis_name="s", num_cores=NC, num_subcores=NS)


def _wid():
    return lax.axis_index("s") * NC + lax.axis_index("c")


CHA2 = 157
GR = 128


def _bcast(nv, lane):
    idx = (jnp.zeros((L,), jnp.int32) + lane)[:, None]
    return lax.gather(
        nv, idx,
        lax.GatherDimensionNumbers(offset_dims=(),
                                   collapsed_slice_dims=(0,),
                                   start_index_map=(0,)),
        slice_sizes=(1,),
        mode=lax.GatherScatterMode.PROMISE_IN_BOUNDS)


def _prep_body(dst_hbm, val_hbm, zf_hbm, x_hbm,
               hist_hbm, feat2_hbm, norm_hbm,
               dst_v, val_v, h_v, norm_v, x_v, xa_v, xb_v, bounce_v,
               sem_a, hist_sh):
    c = lax.axis_index("c")
    s = lax.axis_index("s")
    w = _wid()

    @pl.when(s == 0)
    def _():
        pltpu.sync_copy(zf_hbm, hist_sh)

    pltpu.sync_copy(dst_hbm.at[s], dst_v)
    pltpu.sync_copy(val_hbm, val_v)
    plsc.subcore_barrier()

    pltpu.sync_copy(val_v, hist_sh.at[dst_v], add=True)
    plsc.subcore_barrier()

    base = w * RPW
    pltpu.sync_copy(hist_sh.at[pl.ds(base, RPW)], h_v)

    def nbody(v, carry):
        deg = h_v[pl.ds(v * L, L)] + 1.0
        i = lax.bitcast_convert_type(deg, jnp.int32)
        i = 0x5F3759DF - lax.shift_right_logical(i, 1)
        y = lax.bitcast_convert_type(i, jnp.float32)
        hh = deg * 0.5
        y = y * (1.5 - hh * y * y)
        y = y * (1.5 - hh * y * y)
        y = y * (1.5 - hh * y * y)
        norm_v[pl.ds(v * L, L)] = y
        return carry

    lax.fori_loop(0, RPW // L, nbody, 0)
    pltpu.sync_copy(norm_v, norm_hbm.at[pl.ds(base, RPW)])

    for g in range(RPW // GR):
        row0 = base + g * GR
        pltpu.sync_copy(x_hbm.at[pl.ds(row0 * D, GR * D)], x_v)

        def gbody(g16, carry):
            nv = norm_v[pl.ds(g * GR + g16 * L, L)]
            for lane in range(L):
                nb = _bcast(nv, lane)
                r = g16 * L + lane
                for k in range(D // L):
                    xx = x_v[pl.ds(r * D + k * L, L)] * nb
                    if k < DH // L:
                        xa_v[r, pl.ds(k * L, L)] = xx
                    else:
                        xb_v[r, pl.ds((k - DH // L) * L, L)] = xx
            return carry

        lax.fori_loop(0, GR // L, gbody, 0)
        pltpu.sync_copy(xa_v, feat2_hbm.at[0].at[pl.ds(row0, GR)])
        pltpu.sync_copy(xb_v, feat2_hbm.at[1].at[pl.ds(row0, GR)])

    @pl.when(jnp.logical_and(s == 0, c == 0))
    def _():
        pltpu.sync_copy(hist_sh, bounce_v)
        pltpu.sync_copy(bounce_v, hist_hbm)


def _make_prep():
    return pl.kernel(
        _prep_body,
        out_type=(jax.ShapeDtypeStruct((NP,), jnp.float32),
                  jax.ShapeDtypeStruct((2, NP, DH), jnp.float32),
                  jax.ShapeDtypeStruct((NP,), jnp.float32)),
        mesh=_MESH,
        scratch_types=[
            pltpu.VMEM((CHA2 * 128,), jnp.int32),
            pltpu.VMEM((CHA2 * 128,), jnp.float32),
            pltpu.VMEM((RPW,), jnp.float32),
            pltpu.VMEM((RPW,), jnp.float32),
            pltpu.VMEM((GR * D,), jnp.float32),
            pltpu.VMEM((GR, DH), jnp.float32),
            pltpu.VMEM((GR, DH), jnp.float32),
            pltpu.VMEM((NP,), jnp.float32),
            pltpu.SemaphoreType.DMA,
            pltpu.VMEM_SHARED((NP,), jnp.float32),
        ],
        compiler_params=pltpu.CompilerParams(use_tc_tiling_on_sc=False),
    )


DH = D // 2
NBUF = 4
PF = 2
CHT = 168
RPS = 8


def _agg_body(src_hbm, dst_hbm, feat2_hbm, z_hbm, midx_hbm, vals_hbm,
              zi_hbm, norm_hbm, pout_hbm,
              src_v, dst_v, bufs, sem_g, sem_s, pidx_v, pval_v, idx_v,
              nrm_v, sem_n, agg_sh, dn_sh):
    c = lax.axis_index("c")
    s = lax.axis_index("s")
    w = _wid()

    @pl.when(s == 0)
    def _():
        pltpu.sync_copy(z_hbm, agg_sh)
        pltpu.sync_copy(zi_hbm, dn_sh)

    pltpu.sync_copy(src_hbm.at[s], src_v)
    pltpu.sync_copy(dst_hbm.at[s], dst_v)

    @pl.when(s < NR // RPS)
    def _():
        pltpu.sync_copy(midx_hbm.at[pl.ds(s * RPS * 128, RPS * 128)], pidx_v)
        pltpu.sync_copy(vals_hbm.at[pl.ds(s * RPS * 128, RPS * 128)], pval_v)

    plsc.subcore_barrier()

    @pl.when(s < NR // RPS)
    def _():
        pltpu.sync_copy(pval_v, dn_sh.at[pidx_v])

    plsc.subcore_barrier()

    ftab = feat2_hbm.at[c]
    for n in range(NBUF):
        pltpu.async_copy(ftab.at[src_v.at[n]], bufs[n], sem_g[n])

    def body(t, carry):
        for b in range(NBUF):
            n = NBUF * t + b
            pltpu.make_async_copy(ftab.at[src_v.at[n]], bufs[b],
                                  sem_g[b]).wait()
            pltpu.sync_copy(bufs[b], agg_sh.at[dst_v.at[n]], add=True)

            @pl.when(n + NBUF < CHT)
            def _():
                pltpu.async_copy(ftab.at[src_v.at[n + NBUF]], bufs[b],
                                 sem_g[b])
        return carry

    lax.fori_loop(0, CHT // NBUF, body, 0)
    plsc.subcore_barrier()

    base = w * RPW
    pltpu.sync_copy(dn_sh.at[pl.ds(base, RPW)], idx_v)
    for q in range(RPW // 128):
        iq = idx_v.at[pl.ds(q * 128, 128)]
        g1 = pltpu.async_copy(agg_sh.at[iq], bufs[0], sem_g[0])
        g2 = pltpu.async_copy(norm_hbm.at[iq], nrm_v, sem_n)
        g1.wait()
        g2.wait()

        def sbody(g16, carry):
            nv = nrm_v[pl.ds(g16 * L, L)]
            for lane in range(L):
                nb = _bcast(nv, lane)
                r = g16 * L + lane
                for k in range(DH // L):
                    bufs[0][r, pl.ds(k * L, L)] = (
                        bufs[0][r, pl.ds(k * L, L)] * nb)
            return carry

        lax.fori_loop(0, 128 // L, sbody, 0)
        pltpu.sync_copy(bufs[0], pout_hbm.at[c].at[pl.ds(base + q * 128, 128)])


def _make_agg():
    return pl.kernel(
        _agg_body,
        out_type=jax.ShapeDtypeStruct((NC, NP, DH), jnp.float32),
        mesh=_MESH,
        scratch_types=[
            pltpu.VMEM((CHT, 128), jnp.int32),
            pltpu.VMEM((CHT, 128), jnp.int32),
            [pltpu.VMEM((128, DH), jnp.float32) for _ in range(NBUF)],
            [pltpu.SemaphoreType.DMA for _ in range(NBUF)],
            [pltpu.SemaphoreType.DMA for _ in range(NBUF)],
            pltpu.VMEM((RPS * 128,), jnp.int32),
            pltpu.VMEM((RPS * 128,), jnp.int32),
            pltpu.VMEM((RPW,), jnp.int32),
            pltpu.VMEM((128,), jnp.float32),
            pltpu.SemaphoreType.DMA,
            pltpu.VMEM_SHARED((NP, DH), jnp.float32),
            pltpu.VMEM_SHARED((NP2,), jnp.int32),
        ],
        compiler_params=pltpu.CompilerParams(use_tc_tiling_on_sc=False),
    )


_BD = 512


def _mm_body(p_ref, w_ref, o_ref):
    a = p_ref[...]
    w = w_ref[...]
    dn = (((1,), (1,)), ((), ()))
    o_ref[...] = (
        lax.dot_general(a[0], w[:, :DH], dn,
                        preferred_element_type=jnp.float32) +
        lax.dot_general(a[1], w[:, DH:], dn,
                        preferred_element_type=jnp.float32))


def _matmul(p, w):
    return pl.pallas_call(
        _mm_body,
        grid=(NP // _BD,),
        in_specs=[
            pl.BlockSpec((2, _BD, DH), lambda i: (0, i, 0)),
            pl.BlockSpec((D, D), lambda i: (0, 0)),
        ],
        out_specs=pl.BlockSpec((_BD, D), lambda i: (i, 0)),
        out_shape=jax.ShapeDtypeStruct((NP, D), jnp.float32),
    )(p, w)


NR = NP // 128
NP2 = NP + 128


def _pos_body(h_ref, u_ref, s_ref, o_ref):
    p = (h_ref[...] > 0.0).astype(jnp.float32)
    incl = lax.dot_general(p, u_ref[...], (((1,), (0,)), ((), ())),
                           precision=lax.Precision.HIGHEST,
                           preferred_element_type=jnp.float32)
    rows = incl[:, 127:128]
    rowpref = lax.dot_general(s_ref[...], rows, (((1,), (0,)), ((), ())),
                              precision=lax.Precision.HIGHEST,
                              preferred_element_type=jnp.float32)
    excl = incl - p + rowpref
    col = lax.broadcasted_iota(jnp.int32, (NR, 128), 1)
    o_ref[...] = jnp.where(p > 0.0, excl.astype(jnp.int32), NP + col)


def _positions(h3, u128, s96):
    return pl.pallas_call(
        _pos_body,
        grid=(1,),
        in_specs=[
            pl.BlockSpec((NR, 128), lambda i: (0, 0)),
            pl.BlockSpec((128, 128), lambda i: (0, 0)),
            pl.BlockSpec((NR, NR), lambda i: (0, 0)),
        ],
        out_specs=pl.BlockSpec((NR, 128), lambda i: (0, 0)),
        out_shape=jax.ShapeDtypeStruct((NR, 128), jnp.int32),
    )(h3, u128, s96)


def kernel(x, edge_index, W):
    src = edge_index[0].astype(jnp.int32)
    dst = edge_index[1].astype(jnp.int32)
    x = x.astype(jnp.float32)
    xp = jnp.pad(x, ((0, NP - N), (0, 0)))

    pad_a = (N + (jnp.arange(NS * (CHA2 * 128 - E // NS), dtype=jnp.int32)
                  % (NP - N))).reshape(NS, -1)
    dst_a = jnp.concatenate([dst.reshape(NS, E // NS), pad_a], axis=1)
    val_a = (jnp.arange(CHA2 * 128) < E // NS).astype(jnp.float32)

    loop = jnp.arange(N, dtype=jnp.int32)
    n_pad_c = NS * CHT * 128 - (E + N)
    pad_c = N + (jnp.arange(n_pad_c, dtype=jnp.int32) % (NP - N))
    src_c = jnp.concatenate([src, loop, pad_c]).reshape(NS, CHT, 128)
    dst_c = jnp.concatenate([dst, loop, pad_c]).reshape(NS, CHT, 128)

    zf = jnp.zeros((NP,), jnp.float32)
    z2 = jnp.zeros((NP, DH), jnp.float32)
    zi = jnp.concatenate([
        jnp.zeros((N,), jnp.int32),
        jnp.arange(NP2 - N, dtype=jnp.int32) % N,
    ])
    u128 = jnp.triu(jnp.ones((128, 128), jnp.float32))
    s96 = jnp.tril(jnp.ones((NR, NR), jnp.float32), k=-1)
    vals = jnp.arange(NP, dtype=jnp.int32)

    hist, feat2, norm = _make_prep()(dst_a, val_a, zf, xp.reshape(-1))
    midx = _positions(hist.reshape(NR, 128), u128, s96).reshape(-1)
    part = _make_agg()(src_c, dst_c, feat2, z2, midx, vals, zi, norm)
    fin = _matmul(part, W.astype(jnp.float32))
    return fin[:N]
